# baseline (speedup 1.0000x reference)
import functools

import jax
import jax.numpy as jnp
from jax import lax
from jax.experimental import pallas as pl
from jax.experimental.pallas import tpu as pltpu
from jax.experimental.pallas import tpu_sc as plsc

HIDDEN = 768
EPS = 1e-12
L = 16
NC, NS = 2, 16
NW = NC * NS
G = HIDDEN // L
CH = 32


def _combine_tables(pos_emb, type_emb):
    maxp, hidden = pos_emb.shape
    tv = type_emb.shape[0]

    def body(p_ref, t_ref, o_ref):
        p = p_ref[...]
        for t in range(tv):
            o_ref[t * maxp:(t + 1) * maxp, :] = p + t_ref[t:t + 1, :]

    return pl.pallas_call(
        body,
        out_shape=jax.ShapeDtypeStruct((tv * maxp, hidden), jnp.float32),
    )(pos_emb, type_emb)


def _layernorm_tc(x, gamma, beta):
    tok = x.shape[0]
    bt = 2048

    def body(x_ref, g_ref, b_ref, o_ref):
        xv = x_ref[...]
        mean = jnp.mean(xv, axis=1, keepdims=True)
        cent = xv - mean
        var = jnp.mean(cent * cent, axis=1, keepdims=True)
        o_ref[...] = cent * lax.rsqrt(var + EPS) * g_ref[...] + b_ref[...]

    return pl.pallas_call(
        body,
        grid=(tok // bt,),
        in_specs=[
            pl.BlockSpec((bt, HIDDEN), lambda i: (i, 0)),
            pl.BlockSpec((1, HIDDEN), lambda i: (0, 0)),
            pl.BlockSpec((1, HIDDEN), lambda i: (0, 0)),
        ],
        out_specs=pl.BlockSpec((bt, HIDDEN), lambda i: (i, 0)),
        out_shape=jax.ShapeDtypeStruct((tok, HIDDEN), jnp.float32),
    )(x, gamma.reshape(1, HIDDEN), beta.reshape(1, HIDDEN))


@functools.lru_cache(maxsize=None)
def _sc_gather_sum(tok, maxp):
    tpw = tok // NW
    nch = tpw // CH
    npair = nch // 2
    mesh = plsc.VectorSubcoreMesh(core_axis_name="c", subcore_axis_name="s")

    @functools.partial(
        pl.kernel,
        out_type=jax.ShapeDtypeStruct((tok, HIDDEN), jnp.float32),
        mesh=mesh,
        compiler_params=pltpu.CompilerParams(needs_layout_passes=False),
        scratch_types=[
            pltpu.VMEM((CH,), jnp.int32),
            pltpu.VMEM((CH,), jnp.int32),
            pltpu.VMEM((CH,), jnp.int32),
            pltpu.VMEM((CH,), jnp.int32),
            pltpu.VMEM((CH,), jnp.int32),
            pltpu.VMEM((CH, HIDDEN), jnp.float32),
            pltpu.VMEM((CH, HIDDEN), jnp.float32),
            pltpu.VMEM((CH, HIDDEN), jnp.float32),
            pltpu.VMEM((CH, HIDDEN), jnp.float32),
            pltpu.SemaphoreType.DMA,
            pltpu.SemaphoreType.DMA,
            pltpu.SemaphoreType.DMA,
            pltpu.SemaphoreType.DMA,
        ],
    )
    def k(wid_hbm, pid_hbm, tid_hbm, wtab_hbm, ctab_hbm,
          out_hbm, widxA, widxB, cidxA, cidxB, ttmp,
          wrowsA, wrowsB, crowsA, crowsB,
          semA, semB, osemA, osemB):
        w = lax.axis_index("s") * NC + lax.axis_index("c")
        base = w * tpw

        def load_idx(tb, widx_v, cidx_v):
            pltpu.sync_copy(wid_hbm.at[pl.ds(tb, CH)], widx_v)
            pltpu.sync_copy(pid_hbm.at[pl.ds(tb, CH)], cidx_v)
            pltpu.sync_copy(tid_hbm.at[pl.ds(tb, CH)], ttmp)
            for i in range(CH // L):
                sl = pl.ds(i * L, L)
                cidx_v[sl] = cidx_v[sl] + ttmp[sl] * maxp

        def start_gathers(widx_v, cidx_v, wrows_v, crows_v, sem):
            pltpu.async_copy(wtab_hbm.at[widx_v], wrows_v, sem)
            pltpu.async_copy(ctab_hbm.at[cidx_v], crows_v, sem)

        def wait_gathers(widx_v, cidx_v, wrows_v, crows_v, sem):
            pltpu.make_async_copy(wtab_hbm.at[widx_v], wrows_v, sem).wait()
            pltpu.make_async_copy(ctab_hbm.at[cidx_v], crows_v, sem).wait()

        def start_out(tb, wrows_v, osem):
            pltpu.async_copy(wrows_v, out_hbm.at[pl.ds(tb, CH)], osem)

        def wait_out(wrows_v, osem):
            pltpu.make_async_copy(
                wrows_v, out_hbm.at[pl.ds(0, CH)], osem).wait()

        def compute(wrows_v, crows_v):
            @plsc.parallel_loop(0, CH, step=2)
            def tokpair(t0):
                t1 = t0 + 1
                for g in range(G):
                    sl = pl.ds(g * L, L)
                    wrows_v[t0, sl] = wrows_v[t0, sl] + crows_v[t0, sl]
                    wrows_v[t1, sl] = wrows_v[t1, sl] + crows_v[t1, sl]

        load_idx(base, widxA, cidxA)
        start_gathers(widxA, cidxA, wrowsA, crowsA, semA)

        def pair_body(i, _):
            tb0 = base + (2 * i) * CH
            tb1 = tb0 + CH
            load_idx(tb1, widxB, cidxB)

            @pl.when(i > 0)
            def _():
                wait_out(wrowsB, osemB)
            start_gathers(widxB, cidxB, wrowsB, crowsB, semB)

            wait_gathers(widxA, cidxA, wrowsA, crowsA, semA)
            compute(wrowsA, crowsA)
            start_out(tb0, wrowsA, osemA)

            @pl.when(i < npair - 1)
            def _():
                load_idx(tb0 + 2 * CH, widxA, cidxA)
                wait_out(wrowsA, osemA)
                start_gathers(widxA, cidxA, wrowsA, crowsA, semA)

            wait_gathers(widxB, cidxB, wrowsB, crowsB, semB)
            compute(wrowsB, crowsB)
            start_out(tb1, wrowsB, osemB)
            return 0
        lax.fori_loop(0, npair, pair_body, 0)

        wait_out(wrowsA, osemA)
        wait_out(wrowsB, osemB)

    return k


def kernel(input_ids, token_type_ids, position_ids, word_emb, pos_emb,
           type_emb, gamma, beta):
    b, s = input_ids.shape
    tok = b * s
    maxp = pos_emb.shape[0]
    ctab = _combine_tables(pos_emb, type_emb)
    wid = input_ids.reshape(tok).astype(jnp.int32)
    pid = position_ids.reshape(tok).astype(jnp.int32)
    tid = token_type_ids.reshape(tok).astype(jnp.int32)
    n_split = 4
    step = tok // n_split
    outs = []
    for i in range(n_split):
        sl = slice(i * step, (i + 1) * step)
        sums = _sc_gather_sum(step, maxp)(
            wid[sl], pid[sl], tid[sl], word_emb, ctab)
        outs.append(_layernorm_tc(sums, gamma, beta))
    out = jnp.concatenate(outs, axis=0)
    return out.reshape(b, s, HIDDEN)


# speedup vs baseline: 1.7372x; 1.7372x over previous
import functools

import jax
import jax.numpy as jnp
from jax import lax
from jax.experimental import pallas as pl
from jax.experimental.pallas import tpu as pltpu
from jax.experimental.pallas import tpu_sc as plsc

HIDDEN = 768
EPS = 1e-12
L = 16
NC, NS = 2, 16
NW = NC * NS
G = HIDDEN // L
CH = 32


def _combine_tables(pos_emb, type_emb):
    maxp, hidden = pos_emb.shape
    tv = type_emb.shape[0]

    def body(p_ref, t_ref, o_ref):
        p = p_ref[...]
        for t in range(tv):
            o_ref[t * maxp:(t + 1) * maxp, :] = p + t_ref[t:t + 1, :]

    return pl.pallas_call(
        body,
        out_shape=jax.ShapeDtypeStruct((tv * maxp, hidden), jnp.float32),
    )(pos_emb, type_emb)


def _layernorm_tc(x_packed, gamma, beta):
    tok = x_packed.shape[0]
    h2 = HIDDEN // 2
    bt = 2048

    def body(x_ref, g_ref, b_ref, o_ref):
        xi = x_ref[...]
        lo = lax.bitcast_convert_type(
            lax.shift_left(xi, 16), jnp.float32)
        hi = lax.bitcast_convert_type(
            lax.bitwise_and(xi, jnp.int32(-65536)), jnp.float32)
        mean = (jnp.sum(lo, axis=1, keepdims=True) +
                jnp.sum(hi, axis=1, keepdims=True)) * (1.0 / HIDDEN)
        clo = lo - mean
        chi = hi - mean
        var = (jnp.sum(clo * clo, axis=1, keepdims=True) +
               jnp.sum(chi * chi, axis=1, keepdims=True)) * (1.0 / HIDDEN)
        rinv = lax.rsqrt(var + EPS)
        o_ref[:, :h2] = clo * rinv * g_ref[:, :h2] + b_ref[:, :h2]
        o_ref[:, h2:] = chi * rinv * g_ref[:, h2:] + b_ref[:, h2:]

    return pl.pallas_call(
        body,
        grid=(tok // bt,),
        in_specs=[
            pl.BlockSpec((bt, h2), lambda i: (i, 0)),
            pl.BlockSpec((1, HIDDEN), lambda i: (0, 0)),
            pl.BlockSpec((1, HIDDEN), lambda i: (0, 0)),
        ],
        out_specs=pl.BlockSpec((bt, HIDDEN), lambda i: (i, 0)),
        out_shape=jax.ShapeDtypeStruct((tok, HIDDEN), jnp.float32),
    )(x_packed, gamma.reshape(1, HIDDEN), beta.reshape(1, HIDDEN))


@functools.lru_cache(maxsize=None)
def _sc_gather_sum(tok, maxp):
    tpw = tok // NW
    nch = tpw // CH
    npair = nch // 2
    mesh = plsc.VectorSubcoreMesh(core_axis_name="c", subcore_axis_name="s")

    @functools.partial(
        pl.kernel,
        out_type=jax.ShapeDtypeStruct((tok, HIDDEN // 2), jnp.int32),
        mesh=mesh,
        compiler_params=pltpu.CompilerParams(needs_layout_passes=False),
        scratch_types=[
            pltpu.VMEM((CH,), jnp.int32),
            pltpu.VMEM((CH,), jnp.int32),
            pltpu.VMEM((CH,), jnp.int32),
            pltpu.VMEM((CH,), jnp.int32),
            pltpu.VMEM((CH,), jnp.int32),
            pltpu.VMEM((CH, HIDDEN), jnp.float32),
            pltpu.VMEM((CH, HIDDEN), jnp.float32),
            pltpu.VMEM((CH, HIDDEN), jnp.float32),
            pltpu.VMEM((CH, HIDDEN), jnp.float32),
            pltpu.VMEM((CH, HIDDEN // 2), jnp.int32),
            pltpu.VMEM((CH, HIDDEN // 2), jnp.int32),
            pltpu.SemaphoreType.DMA,
            pltpu.SemaphoreType.DMA,
            pltpu.SemaphoreType.DMA,
            pltpu.SemaphoreType.DMA,
        ],
    )
    def k(wid_hbm, pid_hbm, tid_hbm, wtab_hbm, ctab_hbm,
          out_hbm, widxA, widxB, cidxA, cidxB, ttmp,
          wrowsA, wrowsB, crowsA, crowsB, obufA, obufB,
          semA, semB, osemA, osemB):
        w = lax.axis_index("s") * NC + lax.axis_index("c")
        base = w * tpw

        def load_idx(tb, widx_v, cidx_v):
            pltpu.sync_copy(wid_hbm.at[pl.ds(tb, CH)], widx_v)
            pltpu.sync_copy(pid_hbm.at[pl.ds(tb, CH)], cidx_v)
            pltpu.sync_copy(tid_hbm.at[pl.ds(tb, CH)], ttmp)
            for i in range(CH // L):
                sl = pl.ds(i * L, L)
                cidx_v[sl] = cidx_v[sl] + ttmp[sl] * maxp

        def start_gathers(widx_v, cidx_v, wrows_v, crows_v, sem):
            pltpu.async_copy(wtab_hbm.at[widx_v], wrows_v, sem)
            pltpu.async_copy(ctab_hbm.at[cidx_v], crows_v, sem)

        def wait_gathers(widx_v, cidx_v, wrows_v, crows_v, sem):
            pltpu.make_async_copy(wtab_hbm.at[widx_v], wrows_v, sem).wait()
            pltpu.make_async_copy(ctab_hbm.at[cidx_v], crows_v, sem).wait()

        def start_out(tb, obuf_v, osem):
            pltpu.async_copy(obuf_v, out_hbm.at[pl.ds(tb, CH)], osem)

        def wait_out(obuf_v, osem):
            pltpu.make_async_copy(
                obuf_v, out_hbm.at[pl.ds(0, CH)], osem).wait()

        def compute(wrows_v, crows_v, obuf_v):
            @plsc.parallel_loop(0, CH)
            def tok_body(t):
                for gg in range(G // 2):
                    sl0 = pl.ds(gg * L, L)
                    sl1 = pl.ds((gg + G // 2) * L, L)
                    a = wrows_v[t, sl0] + crows_v[t, sl0]
                    c = wrows_v[t, sl1] + crows_v[t, sl1]
                    packed = plsc.pack(
                        a, c, format=plsc.PackFormat.INTERLEAVED)
                    obuf_v[t, pl.ds(gg * L, L)] = plsc.bitcast(
                        packed, jnp.int32)

        load_idx(base, widxA, cidxA)
        start_gathers(widxA, cidxA, wrowsA, crowsA, semA)

        def pair_body(i, _):
            tb0 = base + (2 * i) * CH
            tb1 = tb0 + CH
            load_idx(tb1, widxB, cidxB)
            start_gathers(widxB, cidxB, wrowsB, crowsB, semB)

            wait_gathers(widxA, cidxA, wrowsA, crowsA, semA)

            @pl.when(i > 0)
            def _():
                wait_out(obufA, osemA)
            compute(wrowsA, crowsA, obufA)
            start_out(tb0, obufA, osemA)

            @pl.when(i < npair - 1)
            def _():
                load_idx(tb0 + 2 * CH, widxA, cidxA)
                start_gathers(widxA, cidxA, wrowsA, crowsA, semA)

            wait_gathers(widxB, cidxB, wrowsB, crowsB, semB)

            @pl.when(i > 0)
            def _():
                wait_out(obufB, osemB)
            compute(wrowsB, crowsB, obufB)
            start_out(tb1, obufB, osemB)
            return 0
        lax.fori_loop(0, npair, pair_body, 0)

        wait_out(obufA, osemA)
        wait_out(obufB, osemB)

    return k


def kernel(input_ids, token_type_ids, position_ids, word_emb, pos_emb,
           type_emb, gamma, beta):
    b, s = input_ids.shape
    tok = b * s
    maxp = pos_emb.shape[0]
    ctab = _combine_tables(pos_emb, type_emb)
    wid = input_ids.reshape(tok).astype(jnp.int32)
    pid = position_ids.reshape(tok).astype(jnp.int32)
    tid = token_type_ids.reshape(tok).astype(jnp.int32)
    sums_packed = _sc_gather_sum(tok, maxp)(wid, pid, tid, word_emb, ctab)
    out = _layernorm_tc(sums_packed, gamma, beta)
    return out.reshape(b, s, HIDDEN)


# speedup vs baseline: 1.8838x; 1.0844x over previous
import functools

import jax
import jax.numpy as jnp
from jax import lax
from jax.experimental import pallas as pl
from jax.experimental.pallas import tpu as pltpu
from jax.experimental.pallas import tpu_sc as plsc

HIDDEN = 768
EPS = 1e-12
L = 16
NC, NS = 2, 16
NW = NC * NS
G = HIDDEN // L
CH = 32


def _combine_tables(pos_emb, type_emb):
    maxp, hidden = pos_emb.shape
    h2 = hidden // 2
    tv = type_emb.shape[0]

    def body(p_ref, t_ref, o_ref):
        p = p_ref[...]
        for t in range(tv):
            c = p + t_ref[t:t + 1, :]
            lo = lax.bitcast_convert_type(
                c[:, :h2].astype(jnp.bfloat16), jnp.uint16
            ).astype(jnp.int32)
            hi = lax.bitcast_convert_type(
                c[:, h2:].astype(jnp.bfloat16), jnp.uint16
            ).astype(jnp.int32)
            o_ref[t * maxp:(t + 1) * maxp, :] = lax.bitwise_or(
                lax.shift_left(hi, 16), lo)

    return pl.pallas_call(
        body,
        out_shape=jax.ShapeDtypeStruct((tv * maxp, h2), jnp.int32),
    )(pos_emb, type_emb)


def _layernorm_tc(x_packed, gamma, beta):
    tok = x_packed.shape[0]
    h2 = HIDDEN // 2
    bt = 2048

    def body(x_ref, g_ref, b_ref, o_ref):
        xi = x_ref[...]
        lo = lax.bitcast_convert_type(
            lax.shift_left(xi, 16), jnp.float32)
        hi = lax.bitcast_convert_type(
            lax.bitwise_and(xi, jnp.int32(-65536)), jnp.float32)
        mean = (jnp.sum(lo, axis=1, keepdims=True) +
                jnp.sum(hi, axis=1, keepdims=True)) * (1.0 / HIDDEN)
        clo = lo - mean
        chi = hi - mean
        var = (jnp.sum(clo * clo, axis=1, keepdims=True) +
               jnp.sum(chi * chi, axis=1, keepdims=True)) * (1.0 / HIDDEN)
        rinv = lax.rsqrt(var + EPS)
        o_ref[:, :h2] = clo * rinv * g_ref[:, :h2] + b_ref[:, :h2]
        o_ref[:, h2:] = chi * rinv * g_ref[:, h2:] + b_ref[:, h2:]

    return pl.pallas_call(
        body,
        grid=(tok // bt,),
        in_specs=[
            pl.BlockSpec((bt, h2), lambda i: (i, 0)),
            pl.BlockSpec((1, HIDDEN), lambda i: (0, 0)),
            pl.BlockSpec((1, HIDDEN), lambda i: (0, 0)),
        ],
        out_specs=pl.BlockSpec((bt, HIDDEN), lambda i: (i, 0)),
        out_shape=jax.ShapeDtypeStruct((tok, HIDDEN), jnp.float32),
    )(x_packed, gamma.reshape(1, HIDDEN), beta.reshape(1, HIDDEN))


@functools.lru_cache(maxsize=None)
def _sc_gather_sum(tok, maxp):
    tpw = tok // NW
    nch = tpw // CH
    npair = nch // 2
    mesh = plsc.VectorSubcoreMesh(core_axis_name="c", subcore_axis_name="s")

    @functools.partial(
        pl.kernel,
        out_type=jax.ShapeDtypeStruct((tok, HIDDEN // 2), jnp.int32),
        mesh=mesh,
        compiler_params=pltpu.CompilerParams(needs_layout_passes=False),
        scratch_types=[
            pltpu.VMEM((CH,), jnp.int32),
            pltpu.VMEM((CH,), jnp.int32),
            pltpu.VMEM((CH,), jnp.int32),
            pltpu.VMEM((CH,), jnp.int32),
            pltpu.VMEM((CH,), jnp.int32),
            pltpu.VMEM((CH, HIDDEN), jnp.float32),
            pltpu.VMEM((CH, HIDDEN), jnp.float32),
            pltpu.VMEM((CH, HIDDEN // 2), jnp.int32),
            pltpu.VMEM((CH, HIDDEN // 2), jnp.int32),
            pltpu.VMEM((CH, HIDDEN // 2), jnp.int32),
            pltpu.VMEM((CH, HIDDEN // 2), jnp.int32),
            pltpu.SemaphoreType.DMA,
            pltpu.SemaphoreType.DMA,
            pltpu.SemaphoreType.DMA,
            pltpu.SemaphoreType.DMA,
        ],
    )
    def k(wid_hbm, pid_hbm, tid_hbm, wtab_hbm, ctab_hbm,
          out_hbm, widxA, widxB, cidxA, cidxB, ttmp,
          wrowsA, wrowsB, crowsA, crowsB, obufA, obufB,
          semA, semB, osemA, osemB):
        w = lax.axis_index("s") * NC + lax.axis_index("c")
        base = w * tpw

        def load_idx(tb, widx_v, cidx_v):
            pltpu.sync_copy(wid_hbm.at[pl.ds(tb, CH)], widx_v)
            pltpu.sync_copy(pid_hbm.at[pl.ds(tb, CH)], cidx_v)
            pltpu.sync_copy(tid_hbm.at[pl.ds(tb, CH)], ttmp)
            for i in range(CH // L):
                sl = pl.ds(i * L, L)
                cidx_v[sl] = cidx_v[sl] + ttmp[sl] * maxp

        def start_gathers(widx_v, cidx_v, wrows_v, crows_v, sem):
            pltpu.async_copy(wtab_hbm.at[widx_v], wrows_v, sem)
            pltpu.async_copy(ctab_hbm.at[cidx_v], crows_v, sem)

        def wait_gathers(widx_v, cidx_v, wrows_v, crows_v, sem):
            pltpu.make_async_copy(wtab_hbm.at[widx_v], wrows_v, sem).wait()
            pltpu.make_async_copy(ctab_hbm.at[cidx_v], crows_v, sem).wait()

        def start_out(tb, obuf_v, osem):
            pltpu.async_copy(obuf_v, out_hbm.at[pl.ds(tb, CH)], osem)

        def wait_out(obuf_v, osem):
            pltpu.make_async_copy(
                obuf_v, out_hbm.at[pl.ds(0, CH)], osem).wait()

        def compute(wrows_v, crows_v, obuf_v):
            @plsc.parallel_loop(0, CH)
            def tok_body(t):
                for gg in range(G // 2):
                    sl = pl.ds(gg * L, L)
                    c32 = crows_v[t, sl]
                    clo = plsc.bitcast(lax.shift_left(c32, 16), jnp.float32)
                    chi = plsc.bitcast(
                        lax.bitwise_and(c32, jnp.int32(-65536)), jnp.float32)
                    a = wrows_v[t, sl] + clo
                    c = wrows_v[t, pl.ds((gg + G // 2) * L, L)] + chi
                    packed = plsc.pack(
                        a, c, format=plsc.PackFormat.INTERLEAVED)
                    obuf_v[t, sl] = plsc.bitcast(packed, jnp.int32)

        load_idx(base, widxA, cidxA)
        start_gathers(widxA, cidxA, wrowsA, crowsA, semA)

        def pair_body(i, _):
            tb0 = base + (2 * i) * CH
            tb1 = tb0 + CH
            load_idx(tb1, widxB, cidxB)
            start_gathers(widxB, cidxB, wrowsB, crowsB, semB)

            wait_gathers(widxA, cidxA, wrowsA, crowsA, semA)

            @pl.when(i > 0)
            def _():
                wait_out(obufA, osemA)
            compute(wrowsA, crowsA, obufA)
            start_out(tb0, obufA, osemA)

            @pl.when(i < npair - 1)
            def _():
                load_idx(tb0 + 2 * CH, widxA, cidxA)
                start_gathers(widxA, cidxA, wrowsA, crowsA, semA)

            wait_gathers(widxB, cidxB, wrowsB, crowsB, semB)

            @pl.when(i > 0)
            def _():
                wait_out(obufB, osemB)
            compute(wrowsB, crowsB, obufB)
            start_out(tb1, obufB, osemB)
            return 0
        lax.fori_loop(0, npair, pair_body, 0)

        wait_out(obufA, osemA)
        wait_out(obufB, osemB)

    return k


def kernel(input_ids, token_type_ids, position_ids, word_emb, pos_emb,
           type_emb, gamma, beta):
    b, s = input_ids.shape
    tok = b * s
    maxp = pos_emb.shape[0]
    ctab = _combine_tables(pos_emb, type_emb)
    wid = input_ids.reshape(tok).astype(jnp.int32)
    pid = position_ids.reshape(tok).astype(jnp.int32)
    tid = token_type_ids.reshape(tok).astype(jnp.int32)
    sums_packed = _sc_gather_sum(tok, maxp)(wid, pid, tid, word_emb, ctab)
    out = _layernorm_tc(sums_packed, gamma, beta)
    return out.reshape(b, s, HIDDEN)


# speedup vs baseline: 1.8858x; 1.0010x over previous
import functools

import jax
import jax.numpy as jnp
from jax import lax
from jax.experimental import pallas as pl
from jax.experimental.pallas import tpu as pltpu
from jax.experimental.pallas import tpu_sc as plsc

HIDDEN = 768
EPS = 1e-12
L = 16
NC, NS = 2, 16
NW = NC * NS
G = HIDDEN // L
CH = 32


def _combine_tables(pos_emb, type_emb):
    maxp, hidden = pos_emb.shape
    h2 = hidden // 2
    tv = type_emb.shape[0]

    def body(p_ref, t_ref, o_ref):
        p = p_ref[...]
        for t in range(tv):
            c = p + t_ref[t:t + 1, :]
            lo = lax.bitcast_convert_type(
                c[:, :h2].astype(jnp.bfloat16), jnp.uint16
            ).astype(jnp.int32)
            hi = lax.bitcast_convert_type(
                c[:, h2:].astype(jnp.bfloat16), jnp.uint16
            ).astype(jnp.int32)
            o_ref[t * maxp:(t + 1) * maxp, :] = lax.bitwise_or(
                lax.shift_left(hi, 16), lo)

    return pl.pallas_call(
        body,
        out_shape=jax.ShapeDtypeStruct((tv * maxp, h2), jnp.int32),
    )(pos_emb, type_emb)


def _layernorm_tc(x_packed, gamma, beta):
    tok = x_packed.shape[0]
    h2 = HIDDEN // 2
    bt = 4096

    def body(x_ref, g_ref, b_ref, o_ref):
        xi = x_ref[...]
        lo = lax.bitcast_convert_type(
            lax.shift_left(xi, 16), jnp.float32)
        hi = lax.bitcast_convert_type(
            lax.bitwise_and(xi, jnp.int32(-65536)), jnp.float32)
        mean = (jnp.sum(lo, axis=1, keepdims=True) +
                jnp.sum(hi, axis=1, keepdims=True)) * (1.0 / HIDDEN)
        clo = lo - mean
        chi = hi - mean
        var = (jnp.sum(clo * clo, axis=1, keepdims=True) +
               jnp.sum(chi * chi, axis=1, keepdims=True)) * (1.0 / HIDDEN)
        rinv = lax.rsqrt(var + EPS)
        o_ref[:, :h2] = clo * rinv * g_ref[:, :h2] + b_ref[:, :h2]
        o_ref[:, h2:] = chi * rinv * g_ref[:, h2:] + b_ref[:, h2:]

    return pl.pallas_call(
        body,
        grid=(tok // bt,),
        in_specs=[
            pl.BlockSpec((bt, h2), lambda i: (i, 0)),
            pl.BlockSpec((1, HIDDEN), lambda i: (0, 0)),
            pl.BlockSpec((1, HIDDEN), lambda i: (0, 0)),
        ],
        out_specs=pl.BlockSpec((bt, HIDDEN), lambda i: (i, 0)),
        out_shape=jax.ShapeDtypeStruct((tok, HIDDEN), jnp.float32),
    )(x_packed, gamma.reshape(1, HIDDEN), beta.reshape(1, HIDDEN))


@functools.lru_cache(maxsize=None)
def _sc_gather_sum(tok, maxp):
    tpw = tok // NW
    nch = tpw // CH
    npair = nch // 2
    mesh = plsc.VectorSubcoreMesh(core_axis_name="c", subcore_axis_name="s")

    @functools.partial(
        pl.kernel,
        out_type=jax.ShapeDtypeStruct((tok, HIDDEN // 2), jnp.int32),
        mesh=mesh,
        compiler_params=pltpu.CompilerParams(needs_layout_passes=False),
        scratch_types=[
            pltpu.VMEM((CH,), jnp.int32),
            pltpu.VMEM((CH,), jnp.int32),
            pltpu.VMEM((CH,), jnp.int32),
            pltpu.VMEM((CH,), jnp.int32),
            pltpu.VMEM((CH,), jnp.int32),
            pltpu.VMEM((CH, HIDDEN), jnp.float32),
            pltpu.VMEM((CH, HIDDEN), jnp.float32),
            pltpu.VMEM((CH, HIDDEN // 2), jnp.int32),
            pltpu.VMEM((CH, HIDDEN // 2), jnp.int32),
            pltpu.VMEM((CH, HIDDEN // 2), jnp.int32),
            pltpu.VMEM((CH, HIDDEN // 2), jnp.int32),
            pltpu.SemaphoreType.DMA,
            pltpu.SemaphoreType.DMA,
            pltpu.SemaphoreType.DMA,
            pltpu.SemaphoreType.DMA,
        ],
    )
    def k(wid_hbm, pid_hbm, tid_hbm, wtab_hbm, ctab_hbm,
          out_hbm, widxA, widxB, cidxA, cidxB, ttmp,
          wrowsA, wrowsB, crowsA, crowsB, obufA, obufB,
          semA, semB, osemA, osemB):
        w = lax.axis_index("s") * NC + lax.axis_index("c")
        base = w * tpw

        def load_idx(tb, widx_v, cidx_v):
            pltpu.sync_copy(wid_hbm.at[pl.ds(tb, CH)], widx_v)
            pltpu.sync_copy(pid_hbm.at[pl.ds(tb, CH)], cidx_v)
            pltpu.sync_copy(tid_hbm.at[pl.ds(tb, CH)], ttmp)
            for i in range(CH // L):
                sl = pl.ds(i * L, L)
                cidx_v[sl] = cidx_v[sl] + ttmp[sl] * maxp

        def start_gathers(widx_v, cidx_v, wrows_v, crows_v, sem):
            pltpu.async_copy(wtab_hbm.at[widx_v], wrows_v, sem)
            pltpu.async_copy(ctab_hbm.at[cidx_v], crows_v, sem)

        def wait_gathers(widx_v, cidx_v, wrows_v, crows_v, sem):
            pltpu.make_async_copy(wtab_hbm.at[widx_v], wrows_v, sem).wait()
            pltpu.make_async_copy(ctab_hbm.at[cidx_v], crows_v, sem).wait()

        def start_out(tb, obuf_v, osem):
            pltpu.async_copy(obuf_v, out_hbm.at[pl.ds(tb, CH)], osem)

        def wait_out(obuf_v, osem):
            pltpu.make_async_copy(
                obuf_v, out_hbm.at[pl.ds(0, CH)], osem).wait()

        def compute(wrows_v, crows_v, obuf_v):
            @plsc.parallel_loop(0, CH)
            def tok_body(t):
                for gg in range(G // 2):
                    sl = pl.ds(gg * L, L)
                    c32 = crows_v[t, sl]
                    clo = plsc.bitcast(lax.shift_left(c32, 16), jnp.float32)
                    chi = plsc.bitcast(
                        lax.bitwise_and(c32, jnp.int32(-65536)), jnp.float32)
                    a = wrows_v[t, sl] + clo
                    c = wrows_v[t, pl.ds((gg + G // 2) * L, L)] + chi
                    packed = plsc.pack(
                        a, c, format=plsc.PackFormat.INTERLEAVED)
                    obuf_v[t, sl] = plsc.bitcast(packed, jnp.int32)

        load_idx(base, widxA, cidxA)
        start_gathers(widxA, cidxA, wrowsA, crowsA, semA)

        def pair_body(i, _):
            tb0 = base + (2 * i) * CH
            tb1 = tb0 + CH
            load_idx(tb1, widxB, cidxB)
            start_gathers(widxB, cidxB, wrowsB, crowsB, semB)

            wait_gathers(widxA, cidxA, wrowsA, crowsA, semA)

            @pl.when(i > 0)
            def _():
                wait_out(obufA, osemA)
            compute(wrowsA, crowsA, obufA)
            start_out(tb0, obufA, osemA)

            @pl.when(i < npair - 1)
            def _():
                load_idx(tb0 + 2 * CH, widxA, cidxA)
                start_gathers(widxA, cidxA, wrowsA, crowsA, semA)

            wait_gathers(widxB, cidxB, wrowsB, crowsB, semB)

            @pl.when(i > 0)
            def _():
                wait_out(obufB, osemB)
            compute(wrowsB, crowsB, obufB)
            start_out(tb1, obufB, osemB)
            return 0
        lax.fori_loop(0, npair, pair_body, 0)

        wait_out(obufA, osemA)
        wait_out(obufB, osemB)

    return k


def kernel(input_ids, token_type_ids, position_ids, word_emb, pos_emb,
           type_emb, gamma, beta):
    b, s = input_ids.shape
    tok = b * s
    maxp = pos_emb.shape[0]
    ctab = _combine_tables(pos_emb, type_emb)
    wid = input_ids.reshape(tok).astype(jnp.int32)
    pid = position_ids.reshape(tok).astype(jnp.int32)
    tid = token_type_ids.reshape(tok).astype(jnp.int32)
    sums_packed = _sc_gather_sum(tok, maxp)(wid, pid, tid, word_emb, ctab)
    out = _layernorm_tc(sums_packed, gamma, beta)
    return out.reshape(b, s, HIDDEN)
